# baseline (speedup 1.0000x reference)
import jax
import jax.numpy as jnp
from jax.experimental import pallas as pl
from jax.experimental.pallas import tpu as pltpu

_N = 4096
_BLK = 512
_NB = _N // _BLK


def _gcn_fused_kernel(x_ref, a_ref, w1_ref, b1_ref, w2_ref, b2_ref,
                      w3_ref, b3_ref, wf_ref, bf_ref, out_ref,
                      a_bf, hx, ns2, h2, h2bf, pmax):
    b = pl.program_id(0)
    i = pl.program_id(1)
    nb = pl.num_programs(0) - 1
    f32 = jnp.float32
    bf16 = jnp.bfloat16
    rows = pl.ds(i * _BLK, _BLK)

    def _linear(v, w_ref, b_ref):
        return jax.lax.dot_general(
            v.astype(bf16), w_ref[:].astype(bf16),
            (((1,), (1,)), ((), ())),
            preferred_element_type=f32) + b_ref[:]

    @pl.when(b >= 1)
    def _layer3_prev():
        ns3 = jnp.dot(a_bf[rows, :], h2bf[:], preferred_element_type=f32)
        h3blk = jax.nn.relu(_linear(h2[rows, :] + ns3, w3_ref, b3_ref))
        m = jnp.max(h3blk.reshape(_BLK // 8, 8, h3blk.shape[1]), axis=0)
        prev = jnp.where(i == 0, jnp.zeros_like(m), pmax[:])
        pmax[:] = jnp.maximum(prev, m)

    @pl.when(b < nb)
    def _current():
        fin = x_ref.shape[2]

        @pl.when(i == 0)
        def _():
            hx[:, :fin] = x_ref[0].astype(bf16)
            hx[:, fin:] = jnp.zeros_like(hx[:, fin:])

        a_blk = a_ref[0].astype(bf16)
        rr = jnp.dot(a_blk, hx[:], preferred_element_type=f32)
        h1blk = jax.nn.relu(_linear(x_ref[0, rows, :] + rr[:, :fin],
                                    w1_ref, b1_ref))
        ns2[rows, :] = rr[:, fin:] + h1blk
        h1blk_bf = h1blk.astype(bf16)
        hx[rows, fin:] = h1blk_bf
        a_bf[rows, :] = a_blk
        ns2[:] = ns2[:] + jnp.dot(
            a_bf[:, pl.ds(i * _BLK, _BLK)], h1blk_bf,
            preferred_element_type=f32)

        @pl.when(i == _NB - 1)
        def _layer2_finish():
            h2new = jax.nn.relu(_linear(ns2[:], w2_ref, b2_ref))
            h2[:] = h2new
            h2bf[:] = h2new.astype(bf16)

    @pl.when((b >= 1) & (i == _NB - 1))
    def _pool_fc():
        pooled = jnp.max(pmax[:], axis=0, keepdims=True)
        out_ref[0] = _linear(pooled, wf_ref, bf_ref)


def kernel(x, edge_index, adjacency, W1, b1, W2, b2, W3, b3, Wf, bf):
    del edge_index
    B, N, Fin = x.shape
    H = W1.shape[0]
    C = Wf.shape[0]

    def a_map(b, i):
        return (jnp.minimum(b, B - 1), jnp.where(b < B, i, _NB - 1), 0)

    def x_map(b, i):
        return (jnp.minimum(b, B - 1), 0, 0)

    grid = (B + 1, _NB)
    out = pl.pallas_call(
        _gcn_fused_kernel,
        grid=grid,
        in_specs=[
            pl.BlockSpec((1, N, Fin), x_map),
            pl.BlockSpec((1, _BLK, N), a_map),
            pl.BlockSpec((H, Fin), lambda b, i: (0, 0)),
            pl.BlockSpec((1, H), lambda b, i: (0, 0)),
            pl.BlockSpec((H, H), lambda b, i: (0, 0)),
            pl.BlockSpec((1, H), lambda b, i: (0, 0)),
            pl.BlockSpec((H, H), lambda b, i: (0, 0)),
            pl.BlockSpec((1, H), lambda b, i: (0, 0)),
            pl.BlockSpec((C, H), lambda b, i: (0, 0)),
            pl.BlockSpec((1, C), lambda b, i: (0, 0)),
        ],
        out_specs=pl.BlockSpec(
            (1, 1, C), lambda b, i: (jnp.maximum(b - 1, 0), 0, 0)),
        out_shape=jax.ShapeDtypeStruct((B, 1, C), jnp.float32),
        scratch_shapes=[
            pltpu.VMEM((N, N), jnp.bfloat16),
            pltpu.VMEM((N, Fin + H), jnp.bfloat16),
            pltpu.VMEM((N, H), jnp.float32),
            pltpu.VMEM((N, H), jnp.float32),
            pltpu.VMEM((N, H), jnp.bfloat16),
            pltpu.VMEM((8, H), jnp.float32),
        ],
        compiler_params=pltpu.CompilerParams(
            dimension_semantics=("arbitrary", "arbitrary"),
            vmem_limit_bytes=63 * 1024 * 1024,
        ),
    )(x, adjacency, W1, b1.reshape(1, H), W2, b2.reshape(1, H),
      W3, b3.reshape(1, H), Wf, bf.reshape(1, C))
    return out.reshape(B, C)


# speedup vs baseline: 1.3735x; 1.3735x over previous
import jax
import jax.numpy as jnp
from jax.experimental import pallas as pl
from jax.experimental.pallas import tpu as pltpu

_N = 4096
_BLK = 512
_NB = _N // _BLK


def _gcn_fused_kernel(x_ref, a_ref, w1_ref, b1_ref, w2_ref, b2_ref,
                      w3_ref, b3_ref, wf_ref, bf_ref, out_ref,
                      a_bf, xbf, h1bf, h2, h2bf, pmax):
    b = pl.program_id(0)
    i = pl.program_id(1)
    nb = pl.num_programs(0) - 1
    f32 = jnp.float32
    bf16 = jnp.bfloat16
    rows = pl.ds(i * _BLK, _BLK)

    def _linear(v, w_ref, b_ref):
        return jax.lax.dot_general(
            v.astype(bf16), w_ref[:].astype(bf16),
            (((1,), (1,)), ((), ())),
            preferred_element_type=f32) + b_ref[:]

    @pl.when(b >= 1)
    def _layer3_prev():
        ns3 = jnp.dot(a_bf[rows, :], h2bf[:], preferred_element_type=f32)
        h3blk = jax.nn.relu(_linear(h2[rows, :] + ns3, w3_ref, b3_ref))
        m = jnp.max(h3blk.reshape(_BLK // 8, 8, h3blk.shape[1]), axis=0)
        prev = jnp.where(i == 0, jnp.zeros_like(m), pmax[:])
        pmax[:] = jnp.maximum(prev, m)

    @pl.when(b < nb)
    def _current():
        @pl.when(i == 0)
        def _():
            xbf[:] = x_ref[0].astype(bf16)

        a_blk = a_ref[0].astype(bf16)
        nsx = jnp.dot(a_blk, xbf[:], preferred_element_type=f32)
        h1blk = jax.nn.relu(_linear(x_ref[0, rows, :] + nsx,
                                    w1_ref, b1_ref))
        h1bf[rows, :] = h1blk.astype(bf16)
        a_bf[rows, :] = a_blk

        @pl.when(i == _NB - 1)
        def _layer2_finish():
            ns2 = jnp.dot(a_bf[:], h1bf[:], preferred_element_type=f32)
            comb = h1bf[:].astype(f32) + ns2
            h2new = jax.nn.relu(_linear(comb, w2_ref, b2_ref))
            h2[:] = h2new
            h2bf[:] = h2new.astype(bf16)

    @pl.when((b >= 1) & (i == _NB - 1))
    def _pool_fc():
        pooled = jnp.max(pmax[:], axis=0, keepdims=True)
        out_ref[0] = _linear(pooled, wf_ref, bf_ref)


def kernel(x, edge_index, adjacency, W1, b1, W2, b2, W3, b3, Wf, bf):
    del edge_index
    B, N, Fin = x.shape
    H = W1.shape[0]
    C = Wf.shape[0]

    def a_map(b, i):
        return (jnp.minimum(b, B - 1), jnp.where(b < B, i, _NB - 1), 0)

    def x_map(b, i):
        return (jnp.minimum(b, B - 1), 0, 0)

    grid = (B + 1, _NB)
    out = pl.pallas_call(
        _gcn_fused_kernel,
        grid=grid,
        in_specs=[
            pl.BlockSpec((1, N, Fin), x_map),
            pl.BlockSpec((1, _BLK, N), a_map),
            pl.BlockSpec((H, Fin), lambda b, i: (0, 0)),
            pl.BlockSpec((1, H), lambda b, i: (0, 0)),
            pl.BlockSpec((H, H), lambda b, i: (0, 0)),
            pl.BlockSpec((1, H), lambda b, i: (0, 0)),
            pl.BlockSpec((H, H), lambda b, i: (0, 0)),
            pl.BlockSpec((1, H), lambda b, i: (0, 0)),
            pl.BlockSpec((C, H), lambda b, i: (0, 0)),
            pl.BlockSpec((1, C), lambda b, i: (0, 0)),
        ],
        out_specs=pl.BlockSpec(
            (1, 1, C), lambda b, i: (jnp.maximum(b - 1, 0), 0, 0)),
        out_shape=jax.ShapeDtypeStruct((B, 1, C), jnp.float32),
        scratch_shapes=[
            pltpu.VMEM((N, N), jnp.bfloat16),
            pltpu.VMEM((N, Fin), jnp.bfloat16),
            pltpu.VMEM((N, H), jnp.bfloat16),
            pltpu.VMEM((N, H), jnp.float32),
            pltpu.VMEM((N, H), jnp.bfloat16),
            pltpu.VMEM((8, H), jnp.float32),
        ],
        compiler_params=pltpu.CompilerParams(
            dimension_semantics=("arbitrary", "arbitrary"),
            vmem_limit_bytes=63 * 1024 * 1024,
        ),
    )(x, adjacency, W1, b1.reshape(1, H), W2, b2.reshape(1, H),
      W3, b3.reshape(1, H), Wf, bf.reshape(1, C))
    return out.reshape(B, C)
